# baseline (speedup 1.0000x reference)
import functools

import jax
import jax.numpy as jnp
from jax import lax
from jax.experimental import pallas as pl
from jax.experimental.pallas import tpu as pltpu
from jax.experimental.pallas import tpu_sc as plsc

N = 50000
E = 800000
D_IN = 128
D_HID = 64
D_OUT = 128

NC = 2
NS = 16
CG = 32
CHUNK = 80
NP = 50048
RPS = NP // NS
ZCH = 136

_MESH = dict(core_axis_name="c", subcore_axis_name="s")


def _fill_const(ref, n_rows, n_cols, value):
    @pl.loop(0, n_rows)
    def _(i):
        for c0 in range(0, n_cols, 16):
            ref[i, pl.ds(c0, 16)] = jnp.full((16,), value, jnp.float32)


BLK = 25
CPS = (E // CHUNK) // NS


def _deg_body(dst2_hbm, out_hbm, didx, ones, zbuf, acc, ssem):
    c = lax.axis_index("c")
    s = lax.axis_index("s")

    @pl.when(c == 0)
    def _():
        _fill_const(ones, CHUNK, 16, 1.0)
        _fill_const(zbuf, ZCH, 16, 0.0)

        @pl.loop(0, RPS // ZCH)
        def _(k):
            pltpu.sync_copy(zbuf, acc.at[pl.ds(s * RPS + k * ZCH, ZCH)])

        plsc.subcore_barrier()

        @pl.loop(0, CPS // BLK)
        def _(r):
            row0 = s * CPS + r * BLK
            pltpu.sync_copy(dst2_hbm.at[pl.ds(row0, BLK)], didx)
            scats = [pltpu.async_copy(ones, acc.at[didx.at[j]], ssem,
                                      add=True)
                     for j in range(BLK)]
            for sc in scats:
                sc.wait()

        plsc.subcore_barrier()
        pltpu.sync_copy(acc.at[pl.ds(s * RPS, RPS)],
                        out_hbm.at[pl.ds(s * RPS, RPS), 0])


def _deg_partials(dst2):
    kern = pl.kernel(
        _deg_body,
        out_type=jax.ShapeDtypeStruct((NP, 8, 16), jnp.float32),
        mesh=plsc.VectorSubcoreMesh(**_MESH),
        scratch_types=[
            pltpu.VMEM((BLK, CHUNK), jnp.int32),
            pltpu.VMEM((CHUNK, 16), jnp.float32),
            pltpu.VMEM((ZCH, 16), jnp.float32),
            pltpu.VMEM_SHARED((NP, 16), jnp.float32),
            pltpu.SemaphoreType.DMA,
        ],
        compiler_params=pltpu.CompilerParams(use_tc_tiling_on_sc=False),
    )
    return kern(dst2)


ABLK = 25
NBUF = 4
LOOKAHEAD = 3


def _agg_body(gpc, src2_hbm, dst2_hbm, g_hbm, out_hbm,
              sidx, didx, rows0, rows1, rows2, rows3, zbuf, acc, gsem, ssem):
    c = lax.axis_index("c")
    s = lax.axis_index("s")
    rows = [rows0, rows1, rows2, rows3]

    _fill_const(zbuf, ZCH, CG, 0.0)

    for g in range(gpc):
        grp = c * gpc + g

        @pl.loop(0, RPS // ZCH)
        def _(k):
            pltpu.sync_copy(zbuf, acc.at[pl.ds(s * RPS + k * ZCH, ZCH)])

        plsc.subcore_barrier()

        @pl.loop(0, CPS // ABLK)
        def _(r):
            row0 = s * CPS + r * ABLK
            pltpu.sync_copy(src2_hbm.at[pl.ds(row0, ABLK)], sidx)
            pltpu.sync_copy(dst2_hbm.at[pl.ds(row0, ABLK)], didx)

            @pl.loop(0, ABLK)
            def _(j):
                for i in range(CHUNK // 16):
                    sidx[j, pl.ds(i * 16, 16)] = (
                        sidx[j, pl.ds(i * 16, 16)] * 4 + grp)

            gathers = [None] * NBUF
            scat = [None] * NBUF
            for j in range(LOOKAHEAD):
                gathers[j % NBUF] = pltpu.async_copy(
                    g_hbm.at[sidx.at[j]], rows[j % NBUF], gsem.at[j % NBUF])
            for j in range(ABLK):
                b = j % NBUF
                jn = j + LOOKAHEAD
                if jn < ABLK:
                    bn = jn % NBUF
                    if scat[bn] is not None:
                        scat[bn].wait()
                        scat[bn] = None
                    gathers[bn] = pltpu.async_copy(
                        g_hbm.at[sidx.at[jn]], rows[bn], gsem.at[bn])
                gathers[b].wait()
                scat[b] = pltpu.async_copy(rows[b], acc.at[didx.at[j]],
                                           ssem.at[b], add=True)
            for t in range(NBUF):
                if scat[t] is not None:
                    scat[t].wait()

        plsc.subcore_barrier()
        pltpu.sync_copy(acc.at[pl.ds(s * RPS, RPS)],
                        out_hbm.at[pl.ds(s * RPS, RPS), grp])


def _aggregate(src2, dst2, g_flat, gpc):
    kern = pl.kernel(
        functools.partial(_agg_body, gpc),
        out_type=jax.ShapeDtypeStruct((NP, 4, CG), jnp.float32),
        mesh=plsc.VectorSubcoreMesh(**_MESH),
        scratch_types=[
            pltpu.VMEM((ABLK, CHUNK), jnp.int32),
            pltpu.VMEM((ABLK, CHUNK), jnp.int32),
            pltpu.VMEM((CHUNK, CG), jnp.float32),
            pltpu.VMEM((CHUNK, CG), jnp.float32),
            pltpu.VMEM((CHUNK, CG), jnp.float32),
            pltpu.VMEM((CHUNK, CG), jnp.float32),
            pltpu.VMEM((ZCH, CG), jnp.float32),
            pltpu.VMEM_SHARED((NP, CG), jnp.float32),
            pltpu.SemaphoreType.DMA((NBUF,)),
            pltpu.SemaphoreType.DMA((NBUF,)),
        ],
        compiler_params=pltpu.CompilerParams(use_tc_tiling_on_sc=False),
    )
    return kern(src2, dst2, g_flat)


_NB = 2048
_HIGH = jax.lax.Precision.HIGHEST


def _mm1_body(x_ref, w_ref, o_ref):
    o_ref[...] = jax.lax.dot(x_ref[...], w_ref[...],
                             precision=_HIGH,
                             preferred_element_type=jnp.float32)


def _mm1(x, W1d):
    return pl.pallas_call(
        _mm1_body,
        grid=(pl.cdiv(N, _NB),),
        in_specs=[pl.BlockSpec((_NB, D_IN), lambda i: (i, 0)),
                  pl.BlockSpec((D_IN, 128), lambda i: (0, 0))],
        out_specs=pl.BlockSpec((_NB, 128), lambda i: (i, 0)),
        out_shape=jax.ShapeDtypeStruct((N, 128), jnp.float32),
    )(x, W1d)


def _dv(degw_block):
    return jax.lax.rsqrt(degw_block[:, 0:1] + 1.0)


def _scale_body(h_ref, degw_ref, g_ref):
    g_ref[...] = h_ref[...] * _dv(degw_ref[...])


def _scale(h1d, degw):
    return pl.pallas_call(
        _scale_body,
        grid=(pl.cdiv(N, _NB),),
        in_specs=[pl.BlockSpec((_NB, 128), lambda i: (i, 0)),
                  pl.BlockSpec((_NB, 128), lambda i: (i, 0))],
        out_specs=pl.BlockSpec((_NB, 128), lambda i: (i, 0)),
        out_shape=jax.ShapeDtypeStruct((N, 128), jnp.float32),
    )(h1d, degw)


def _mid_body(agg_ref, g1_ref, degw_ref, w2_ref, b1_ref, g2_ref):
    dv = _dv(degw_ref[...])
    a = agg_ref[:, :D_HID] + g1_ref[:, :D_HID]
    h = jnp.maximum(a * dv + b1_ref[...][None, :], 0.0)
    g2_ref[...] = jax.lax.dot(h, w2_ref[...], precision=_HIGH,
                              preferred_element_type=jnp.float32) * dv


def _mid(agg1w, g1w, degw, W2, b1):
    return pl.pallas_call(
        _mid_body,
        grid=(pl.cdiv(N, _NB),),
        in_specs=[pl.BlockSpec((_NB, 128), lambda i: (i, 0)),
                  pl.BlockSpec((_NB, 128), lambda i: (i, 0)),
                  pl.BlockSpec((_NB, 128), lambda i: (i, 0)),
                  pl.BlockSpec((D_HID, D_OUT), lambda i: (0, 0)),
                  pl.BlockSpec((D_HID,), lambda i: (0,))],
        out_specs=pl.BlockSpec((_NB, 128), lambda i: (i, 0)),
        out_shape=jax.ShapeDtypeStruct((N, 128), jnp.float32),
    )(agg1w, g1w, degw, W2, b1)


def _final_body(agg_ref, g2_ref, degw_ref, b2_ref, o_ref):
    dv = _dv(degw_ref[...])
    o_ref[...] = (agg_ref[...] + g2_ref[...]) * dv + b2_ref[...][None, :]


def _final(agg2w, g2w, degw, b2):
    return pl.pallas_call(
        _final_body,
        grid=(pl.cdiv(N, _NB),),
        in_specs=[pl.BlockSpec((_NB, 128), lambda i: (i, 0)),
                  pl.BlockSpec((_NB, 128), lambda i: (i, 0)),
                  pl.BlockSpec((_NB, 128), lambda i: (i, 0)),
                  pl.BlockSpec((D_OUT,), lambda i: (0,))],
        out_specs=pl.BlockSpec((_NB, D_OUT), lambda i: (i, 0)),
        out_shape=jax.ShapeDtypeStruct((N, D_OUT), jnp.float32),
    )(agg2w, g2w, degw, b2)


@jax.jit
def _run(x, edge_index, W1, b1, W2, b2):
    src2 = edge_index[0].reshape(E // CHUNK, CHUNK)
    dst2 = edge_index[1].reshape(E // CHUNK, CHUNK)
    W1d = jnp.concatenate([W1, W1], axis=1)

    degp = _deg_partials(dst2)
    degw = degp.reshape(NP, 128)
    h1d = _mm1(x, W1d)
    g1w = _scale(h1d, degw)
    agg1 = _aggregate(src2, dst2, g1w.reshape(4 * N, CG), 1)
    g2w = _mid(agg1.reshape(NP, 128), g1w, degw, W2, b1)
    agg2 = _aggregate(src2, dst2, g2w.reshape(4 * N, CG), 2)
    return _final(agg2.reshape(NP, 128), g2w, degw, b2)


def kernel(x, edge_index, W1, b1, W2, b2):
    assert x.shape == (N, D_IN) and edge_index.shape == (2, E)
    return _run(x, edge_index, W1, b1, W2, b2)
